# Initial kernel scaffold; baseline (speedup 1.0000x reference)
import jax
import jax.numpy as jnp
from jax.experimental import pallas as pl


def kernel(x, edge_index, edge_weight, W1, b1, W2, b2, W_ih, W_hh, b_ih, b_hh, fcW, fcb):
    raise NotImplementedError("write your pallas kernel here")


# speedup vs baseline: 3.3035x; 3.3035x over previous
import jax
import jax.numpy as jnp
from jax import lax
from jax.experimental import pallas as pl
from jax.experimental.pallas import tpu as pltpu
from jax.experimental.pallas import tpu_sc as plsc

N_, T_, F_, H_, E_, O_ = 10000, 12, 128, 128, 320000, 2
NC, NS = 2, 16
NW = NC * NS
NP = 10240
NQ = 4
HQ = H_ // NQ
K_ = 128
EPS = E_ // NS
NCH2 = -(-EPS // K_)
EPP2 = NCH2 * K_
RPS = NP // NS
RZB = 128
RB = 1024
NB = NP // RB
RBG = 512
NBG = NP // RBG
MSK = (1 << 14) - 1


def _deg_body(pk, ewp, out, pk_v, ew_v, deg_v):
    c = lax.axis_index("c")
    s = lax.axis_index("s")
    wid = s * NC + c
    pltpu.sync_copy(pk.at[s], pk_v)
    pltpu.sync_copy(ewp.at[s], ew_v)

    def zero(i, _):
        deg_v[pl.ds(i * 16, 16)] = jnp.zeros((16,), jnp.float32)
        return 0

    lax.fori_loop(0, NP // 16, zero, 0)

    def chunk(jj, _):
        j = jj * NC + c
        for q in range(8):
            sl = pl.ds(q * 16, 16)
            d = lax.shift_right_logical(pk_v[j, sl], 14)
            plsc.addupdate_scatter(deg_v, [d], ew_v[j, sl])
        return 0

    lax.fori_loop(0, (NCH2 + 1 - c) // NC, chunk, 0)
    pltpu.sync_copy(deg_v, out.at[pl.ds(wid * NP, NP)])


def _spmm_body(table, pk, ewp, out, pk_v, dst_v, ew_v, idx_v,
               rows_v, zero_v, acc, sem):
    c = lax.axis_index("c")
    s = lax.axis_index("s")
    pltpu.sync_copy(pk.at[s], pk_v)
    pltpu.sync_copy(ewp.at[s], ew_v)

    def upk(j, _):
        for q in range(8):
            sl = pl.ds(q * 16, 16)
            dst_v[j, sl] = lax.shift_right_logical(pk_v[j, sl], 14)
        return 0

    lax.fori_loop(0, NCH2, upk, 0)

    def zb(i, _):
        for q in range(HQ // 16):
            zero_v[i, pl.ds(q * 16, 16)] = jnp.zeros((16,), jnp.float32)
        return 0

    lax.fori_loop(0, RZB, zb, 0)
    for i in range(RPS // RZB):
        pltpu.sync_copy(zero_v, acc.at[pl.ds(s * RPS + i * RZB, RZB)])

    def u_body(u, _):
        t = u // 2
        qid = c + 2 * (u % 2)
        off = qid * (T_ * NP) + t * NP

        def ib(j, _):
            for q in range(8):
                sl = pl.ds(q * 16, 16)
                idx_v[j, sl] = (pk_v[j, sl] & MSK) + off
            return 0

        lax.fori_loop(0, NCH2, ib, 0)
        plsc.subcore_barrier()

        def chunk(j, _):
            pltpu.async_copy(table.at[idx_v.at[j]], rows_v, sem).wait()

            def scale(e, _):
                nv = plsc.load_gather(
                    ew_v, [jnp.full((16,), j, jnp.int32),
                           jnp.full((16,), e, jnp.int32)])
                for q in range(HQ // 16):
                    sl = pl.ds(q * 16, 16)
                    rows_v[e, sl] = rows_v[e, sl] * nv
                return 0

            lax.fori_loop(0, K_, scale, 0)
            pltpu.sync_copy(rows_v, acc.at[dst_v.at[j]], add=True)
            return 0

        lax.fori_loop(0, NCH2, chunk, 0)
        plsc.subcore_barrier()
        for i in range(RPS // RZB):
            sl = pl.ds(s * RPS + i * RZB, RZB)
            pltpu.sync_copy(acc.at[sl], out.at[qid].at[t].at[sl])
            pltpu.sync_copy(zero_v, acc.at[sl])
        return 0

    lax.fori_loop(0, T_ * 2, u_body, 0)


def _dinv_tc(parts_ref, o_ref):
    deg = jnp.sum(parts_ref[...], axis=0) + 1.0
    safe = jnp.where(deg > 0, deg, 1.0)
    o_ref[...] = jnp.where(deg > 0, lax.rsqrt(safe), 0.0)


def _mm1_tc(x_ref, w_ref, dinv_ref, o_ref):
    z = lax.dot_general(x_ref[...], w_ref[...],
                        (((1,), (1,)), ((), ())),
                        preferred_element_type=jnp.float32)
    z = z * dinv_ref[0, 0, :][:, None]
    for q in range(NQ):
        o_ref[q] = z[:, q * HQ:(q + 1) * HQ]


def _mm2_tc(s_ref, z_ref, dinv_ref, b_ref, w_ref, o_ref):
    d = dinv_ref[0, 0, :][:, None]
    pre = jnp.concatenate(
        [s_ref[q] + z_ref[q] for q in range(NQ)], axis=1)
    h = jnp.maximum(d * pre + b_ref[...][None, :], 0.0)
    z = d * lax.dot_general(h, w_ref[...], (((1,), (1,)), ((), ())),
                            preferred_element_type=jnp.float32)
    for q in range(NQ):
        o_ref[q] = z[:, q * HQ:(q + 1) * HQ]


def _gru_tc(s_ref, z_ref, dinv_ref, b2_ref,
            wih_ref, whh_ref, bih_ref, bhh_ref, fcw_ref, fcb_ref, o_ref):
    d = dinv_ref[0, 0, :][:, None]
    b2 = b2_ref[...][None, :]
    bih = bih_ref[...][None, :]
    bhh = bhh_ref[...][None, :]
    wih = wih_ref[...]
    whh = whh_ref[...]
    h = jnp.zeros((RBG, H_), jnp.float32)
    for t in range(T_):
        pre = jnp.concatenate(
            [s_ref[q, t] + z_ref[q, t] for q in range(NQ)], axis=1)
        h2 = jnp.maximum(d * pre + b2, 0.0)
        gi = lax.dot_general(h2, wih, (((1,), (1,)), ((), ())),
                             preferred_element_type=jnp.float32) + bih
        gh = lax.dot_general(h, whh, (((1,), (1,)), ((), ())),
                             preferred_element_type=jnp.float32) + bhh
        r = jax.nn.sigmoid(gi[:, :H_] + gh[:, :H_])
        zg = jax.nn.sigmoid(gi[:, H_:2 * H_] + gh[:, H_:2 * H_])
        ng = jnp.tanh(gi[:, 2 * H_:] + r * gh[:, 2 * H_:])
        h = (1.0 - zg) * ng + zg * h
    preds = lax.dot_general(h, fcw_ref[...], (((1,), (1,)), ((), ())),
                            preferred_element_type=jnp.float32)
    preds = preds + fcb_ref[...][None, :]
    mu = jax.nn.sigmoid(preds[:, 0:1])
    sg = jax.nn.softplus(preds[:, 1:2]) + 1e-6
    o_ref[...] = jnp.concatenate([mu, sg], axis=1)


def kernel(x, edge_index, edge_weight, W1, b1, W2, b2,
           W_ih, W_hh, b_ih, b_hh, fcW, fcb):
    src = edge_index[0].astype(jnp.int32)
    dst = edge_index[1].astype(jnp.int32)
    ew = edge_weight.astype(jnp.float32)
    packed = src | (dst << 14)
    spad = EPP2 - EPS
    pk2 = jnp.pad(packed.reshape(NS, EPS), ((0, 0), (0, spad))).reshape(NS, NCH2, K_)
    ewp2 = jnp.pad(ew.reshape(NS, EPS), ((0, 0), (0, spad))).reshape(NS, NCH2, K_)
    xT = jnp.pad(jnp.transpose(x, (1, 0, 2)),
                 ((0, 0), (0, NP - N_), (0, 0))).reshape(T_ * NP, F_)

    mesh = plsc.VectorSubcoreMesh(core_axis_name="c", subcore_axis_name="s")
    deg_fn = pl.kernel(
        _deg_body,
        out_type=jax.ShapeDtypeStruct((NW * NP,), jnp.float32),
        mesh=mesh,
        compiler_params=pltpu.CompilerParams(needs_layout_passes=False,
                                             use_tc_tiling_on_sc=False),
        scratch_types=[
            pltpu.VMEM((NCH2, K_), jnp.int32),
            pltpu.VMEM((NCH2, K_), jnp.float32),
            pltpu.VMEM((NP,), jnp.float32),
        ],
    )
    degp = deg_fn(pk2, ewp2)

    dinv = pl.pallas_call(
        _dinv_tc,
        out_shape=jax.ShapeDtypeStruct((NP,), jnp.float32),
    )(degp.reshape(NW, NP))
    dinv3 = dinv.reshape(NB, 1, RB)

    z1s = pl.pallas_call(
        _mm1_tc,
        grid=(T_ * NP // RB,),
        in_specs=[
            pl.BlockSpec((RB, F_), lambda i: (i, 0)),
            pl.BlockSpec((H_, F_), lambda i: (0, 0)),
            pl.BlockSpec((1, 1, RB), lambda i: (i % NB, 0, 0)),
        ],
        out_specs=pl.BlockSpec((NQ, RB, HQ), lambda i: (0, i, 0)),
        out_shape=jax.ShapeDtypeStruct((NQ, T_ * NP, HQ), jnp.float32),
    )(xT, W1, dinv3)

    spmm_fn = pl.kernel(
        _spmm_body,
        out_type=jax.ShapeDtypeStruct((NQ, T_, NP, HQ), jnp.float32),
        mesh=mesh,
        compiler_params=pltpu.CompilerParams(needs_layout_passes=False,
                                             use_tc_tiling_on_sc=False),
        scratch_types=[
            pltpu.VMEM((NCH2, K_), jnp.int32),
            pltpu.VMEM((NCH2, K_), jnp.int32),
            pltpu.VMEM((NCH2, K_), jnp.float32),
            pltpu.VMEM((NCH2, K_), jnp.int32),
            pltpu.VMEM((K_, HQ), jnp.float32),
            pltpu.VMEM((RZB, HQ), jnp.float32),
            pltpu.VMEM_SHARED((NP, HQ), jnp.float32),
            pltpu.SemaphoreType.DMA,
        ],
    )

    s1 = spmm_fn(z1s.reshape(NQ * T_ * NP, HQ), pk2, ewp2)

    z2s = pl.pallas_call(
        _mm2_tc,
        grid=(T_ * NP // RB,),
        in_specs=[
            pl.BlockSpec((NQ, RB, HQ), lambda i: (0, i, 0)),
            pl.BlockSpec((NQ, RB, HQ), lambda i: (0, i, 0)),
            pl.BlockSpec((1, 1, RB), lambda i: (i % NB, 0, 0)),
            pl.BlockSpec((H_,), lambda i: (0,)),
            pl.BlockSpec((H_, H_), lambda i: (0, 0)),
        ],
        out_specs=pl.BlockSpec((NQ, RB, HQ), lambda i: (0, i, 0)),
        out_shape=jax.ShapeDtypeStruct((NQ, T_ * NP, HQ), jnp.float32),
    )(s1.reshape(NQ, T_ * NP, HQ), z1s, dinv3, b1, W2)

    s2 = spmm_fn(z2s.reshape(NQ * T_ * NP, HQ), pk2, ewp2)

    dinv_g = dinv.reshape(NBG, 1, RBG)
    out = pl.pallas_call(
        _gru_tc,
        grid=(NBG,),
        in_specs=[
            pl.BlockSpec((NQ, T_, RBG, HQ), lambda i: (0, 0, i, 0)),
            pl.BlockSpec((NQ, T_, RBG, HQ), lambda i: (0, 0, i, 0)),
            pl.BlockSpec((1, 1, RBG), lambda i: (i, 0, 0)),
            pl.BlockSpec((H_,), lambda i: (0,)),
            pl.BlockSpec((3 * H_, H_), lambda i: (0, 0)),
            pl.BlockSpec((3 * H_, H_), lambda i: (0, 0)),
            pl.BlockSpec((3 * H_,), lambda i: (0,)),
            pl.BlockSpec((3 * H_,), lambda i: (0,)),
            pl.BlockSpec((O_, H_), lambda i: (0, 0)),
            pl.BlockSpec((O_,), lambda i: (0,)),
        ],
        out_specs=pl.BlockSpec((RBG, O_), lambda i: (i, 0)),
        out_shape=jax.ShapeDtypeStruct((NP, O_), jnp.float32),
    )(s2, z2s.reshape(NQ, T_, NP, HQ),
      dinv_g, b2, W_ih, W_hh, b_ih, b_hh, fcW, fcb)

    return out[:N_, 0], out[:N_, 1]


# speedup vs baseline: 4.3688x; 1.3225x over previous
import jax
import jax.numpy as jnp
from jax import lax
from jax.experimental import pallas as pl
from jax.experimental.pallas import tpu as pltpu
from jax.experimental.pallas import tpu_sc as plsc

N_, T_, F_, H_, E_, O_ = 10000, 12, 128, 128, 320000, 2
NC, NS = 2, 16
NW = NC * NS
NP = 10240
NQ = 4
HQ = H_ // NQ
K_ = 128
EPS = E_ // NS
NCH2 = -(-EPS // K_)
EPP2 = NCH2 * K_
RPS = NP // NS
RZB = 128
RB = 1024
NB = NP // RB
RBG = 512
NBG = NP // RBG
MSK = (1 << 14) - 1


def _deg_body(pk, ewp, out, pk_v, ew_v, deg_v):
    c = lax.axis_index("c")
    s = lax.axis_index("s")
    wid = s * NC + c
    pltpu.sync_copy(pk.at[s], pk_v)
    pltpu.sync_copy(ewp.at[s], ew_v)

    def zero(i, _):
        deg_v[pl.ds(i * 16, 16)] = jnp.zeros((16,), jnp.float32)
        return 0

    lax.fori_loop(0, NP // 16, zero, 0)

    def chunk(jj, _):
        j = jj * NC + c
        for q in range(8):
            sl = pl.ds(q * 16, 16)
            d = lax.shift_right_logical(pk_v[j, sl], 14)
            plsc.addupdate_scatter(deg_v, [d], ew_v[j, sl])
        return 0

    lax.fori_loop(0, (NCH2 + 1 - c) // NC, chunk, 0)
    pltpu.sync_copy(deg_v, out.at[pl.ds(wid * NP, NP)])


def _spmm_body(table, pk, ewp, out, pk_v, dst_v, ew_v, idx_v,
               rows_v, zero_v, acc, sem):
    c = lax.axis_index("c")
    s = lax.axis_index("s")
    pltpu.sync_copy(pk.at[s], pk_v)
    pltpu.sync_copy(ewp.at[s], ew_v)

    @plsc.parallel_loop(0, NCH2, 1, unroll=4)
    def upk(j):
        for q in range(8):
            sl = pl.ds(q * 16, 16)
            dst_v[j, sl] = lax.shift_right_logical(pk_v[j, sl], 14)

    def zb(i, _):
        for q in range(HQ // 16):
            zero_v[i, pl.ds(q * 16, 16)] = jnp.zeros((16,), jnp.float32)
        return 0

    lax.fori_loop(0, RZB, zb, 0)
    for i in range(RPS // RZB):
        pltpu.sync_copy(zero_v, acc.at[pl.ds(s * RPS + i * RZB, RZB)])

    def u_body(u, _):
        t = u // 2
        qid = c + 2 * (u % 2)
        off = qid * (T_ * NP) + t * NP

        @plsc.parallel_loop(0, NCH2, 1, unroll=4)
        def ib(j):
            for q in range(8):
                sl = pl.ds(q * 16, 16)
                idx_v[j, sl] = (pk_v[j, sl] & MSK) + off
        plsc.subcore_barrier()

        def chunk(j, _):
            pltpu.async_copy(table.at[idx_v.at[j]], rows_v, sem).wait()

            @plsc.parallel_loop(0, K_, 1, unroll=8)
            def scale(e):
                nv = plsc.load_gather(
                    ew_v, [jnp.full((16,), j, jnp.int32),
                           jnp.full((16,), e, jnp.int32)])
                for q in range(HQ // 16):
                    sl = pl.ds(q * 16, 16)
                    rows_v[e, sl] = rows_v[e, sl] * nv

            pltpu.sync_copy(rows_v, acc.at[dst_v.at[j]], add=True)
            return 0

        lax.fori_loop(0, NCH2, chunk, 0)
        plsc.subcore_barrier()
        for i in range(RPS // RZB):
            sl = pl.ds(s * RPS + i * RZB, RZB)
            pltpu.sync_copy(acc.at[sl], out.at[qid].at[t].at[sl])
            pltpu.sync_copy(zero_v, acc.at[sl])
        return 0

    lax.fori_loop(0, T_ * 2, u_body, 0)


def _dinv_tc(parts_ref, o_ref):
    deg = jnp.sum(parts_ref[...], axis=0) + 1.0
    safe = jnp.where(deg > 0, deg, 1.0)
    o_ref[...] = jnp.where(deg > 0, lax.rsqrt(safe), 0.0)


def _mm1_tc(x_ref, w_ref, dinv_ref, o_ref):
    z = lax.dot_general(x_ref[...], w_ref[...],
                        (((1,), (1,)), ((), ())),
                        preferred_element_type=jnp.float32)
    z = z * dinv_ref[0, 0, :][:, None]
    for q in range(NQ):
        o_ref[q] = z[:, q * HQ:(q + 1) * HQ]


def _mm2_tc(s_ref, z_ref, dinv_ref, b_ref, w_ref, o_ref):
    d = dinv_ref[0, 0, :][:, None]
    pre = jnp.concatenate(
        [s_ref[q] + z_ref[q] for q in range(NQ)], axis=1)
    h = jnp.maximum(d * pre + b_ref[...][None, :], 0.0)
    z = d * lax.dot_general(h, w_ref[...], (((1,), (1,)), ((), ())),
                            preferred_element_type=jnp.float32)
    for q in range(NQ):
        o_ref[q] = z[:, q * HQ:(q + 1) * HQ]


def _gru_tc(s_ref, z_ref, dinv_ref, b2_ref,
            wih_ref, whh_ref, bih_ref, bhh_ref, fcw_ref, fcb_ref, o_ref):
    d = dinv_ref[0, 0, :][:, None]
    b2 = b2_ref[...][None, :]
    bih = bih_ref[...][None, :]
    bhh = bhh_ref[...][None, :]
    wih = wih_ref[...]
    whh = whh_ref[...]
    h = jnp.zeros((RBG, H_), jnp.float32)
    for t in range(T_):
        pre = jnp.concatenate(
            [s_ref[q, t] + z_ref[q, t] for q in range(NQ)], axis=1)
        h2 = jnp.maximum(d * pre + b2, 0.0)
        gi = lax.dot_general(h2, wih, (((1,), (1,)), ((), ())),
                             preferred_element_type=jnp.float32) + bih
        gh = lax.dot_general(h, whh, (((1,), (1,)), ((), ())),
                             preferred_element_type=jnp.float32) + bhh
        r = jax.nn.sigmoid(gi[:, :H_] + gh[:, :H_])
        zg = jax.nn.sigmoid(gi[:, H_:2 * H_] + gh[:, H_:2 * H_])
        ng = jnp.tanh(gi[:, 2 * H_:] + r * gh[:, 2 * H_:])
        h = (1.0 - zg) * ng + zg * h
    preds = lax.dot_general(h, fcw_ref[...], (((1,), (1,)), ((), ())),
                            preferred_element_type=jnp.float32)
    preds = preds + fcb_ref[...][None, :]
    mu = jax.nn.sigmoid(preds[:, 0:1])
    sg = jax.nn.softplus(preds[:, 1:2]) + 1e-6
    o_ref[...] = jnp.concatenate([mu, sg], axis=1)


def kernel(x, edge_index, edge_weight, W1, b1, W2, b2,
           W_ih, W_hh, b_ih, b_hh, fcW, fcb):
    src = edge_index[0].astype(jnp.int32)
    dst = edge_index[1].astype(jnp.int32)
    ew = edge_weight.astype(jnp.float32)
    packed = src | (dst << 14)
    spad = EPP2 - EPS
    pk2 = jnp.pad(packed.reshape(NS, EPS), ((0, 0), (0, spad))).reshape(NS, NCH2, K_)
    ewp2 = jnp.pad(ew.reshape(NS, EPS), ((0, 0), (0, spad))).reshape(NS, NCH2, K_)
    xT = jnp.pad(jnp.transpose(x, (1, 0, 2)),
                 ((0, 0), (0, NP - N_), (0, 0))).reshape(T_ * NP, F_)

    mesh = plsc.VectorSubcoreMesh(core_axis_name="c", subcore_axis_name="s")
    deg_fn = pl.kernel(
        _deg_body,
        out_type=jax.ShapeDtypeStruct((NW * NP,), jnp.float32),
        mesh=mesh,
        compiler_params=pltpu.CompilerParams(needs_layout_passes=False,
                                             use_tc_tiling_on_sc=False),
        scratch_types=[
            pltpu.VMEM((NCH2, K_), jnp.int32),
            pltpu.VMEM((NCH2, K_), jnp.float32),
            pltpu.VMEM((NP,), jnp.float32),
        ],
    )
    degp = deg_fn(pk2, ewp2)

    dinv = pl.pallas_call(
        _dinv_tc,
        out_shape=jax.ShapeDtypeStruct((NP,), jnp.float32),
    )(degp.reshape(NW, NP))
    dinv3 = dinv.reshape(NB, 1, RB)

    z1s = pl.pallas_call(
        _mm1_tc,
        grid=(T_ * NP // RB,),
        in_specs=[
            pl.BlockSpec((RB, F_), lambda i: (i, 0)),
            pl.BlockSpec((H_, F_), lambda i: (0, 0)),
            pl.BlockSpec((1, 1, RB), lambda i: (i % NB, 0, 0)),
        ],
        out_specs=pl.BlockSpec((NQ, RB, HQ), lambda i: (0, i, 0)),
        out_shape=jax.ShapeDtypeStruct((NQ, T_ * NP, HQ), jnp.float32),
    )(xT, W1, dinv3)

    spmm_fn = pl.kernel(
        _spmm_body,
        out_type=jax.ShapeDtypeStruct((NQ, T_, NP, HQ), jnp.float32),
        mesh=mesh,
        compiler_params=pltpu.CompilerParams(needs_layout_passes=False,
                                             use_tc_tiling_on_sc=False),
        scratch_types=[
            pltpu.VMEM((NCH2, K_), jnp.int32),
            pltpu.VMEM((NCH2, K_), jnp.int32),
            pltpu.VMEM((NCH2, K_), jnp.float32),
            pltpu.VMEM((NCH2, K_), jnp.int32),
            pltpu.VMEM((K_, HQ), jnp.float32),
            pltpu.VMEM((RZB, HQ), jnp.float32),
            pltpu.VMEM_SHARED((NP, HQ), jnp.float32),
            pltpu.SemaphoreType.DMA,
        ],
    )

    s1 = spmm_fn(z1s.reshape(NQ * T_ * NP, HQ), pk2, ewp2)

    z2s = pl.pallas_call(
        _mm2_tc,
        grid=(T_ * NP // RB,),
        in_specs=[
            pl.BlockSpec((NQ, RB, HQ), lambda i: (0, i, 0)),
            pl.BlockSpec((NQ, RB, HQ), lambda i: (0, i, 0)),
            pl.BlockSpec((1, 1, RB), lambda i: (i % NB, 0, 0)),
            pl.BlockSpec((H_,), lambda i: (0,)),
            pl.BlockSpec((H_, H_), lambda i: (0, 0)),
        ],
        out_specs=pl.BlockSpec((NQ, RB, HQ), lambda i: (0, i, 0)),
        out_shape=jax.ShapeDtypeStruct((NQ, T_ * NP, HQ), jnp.float32),
    )(s1.reshape(NQ, T_ * NP, HQ), z1s, dinv3, b1, W2)

    s2 = spmm_fn(z2s.reshape(NQ * T_ * NP, HQ), pk2, ewp2)

    dinv_g = dinv.reshape(NBG, 1, RBG)
    out = pl.pallas_call(
        _gru_tc,
        grid=(NBG,),
        in_specs=[
            pl.BlockSpec((NQ, T_, RBG, HQ), lambda i: (0, 0, i, 0)),
            pl.BlockSpec((NQ, T_, RBG, HQ), lambda i: (0, 0, i, 0)),
            pl.BlockSpec((1, 1, RBG), lambda i: (i, 0, 0)),
            pl.BlockSpec((H_,), lambda i: (0,)),
            pl.BlockSpec((3 * H_, H_), lambda i: (0, 0)),
            pl.BlockSpec((3 * H_, H_), lambda i: (0, 0)),
            pl.BlockSpec((3 * H_,), lambda i: (0,)),
            pl.BlockSpec((3 * H_,), lambda i: (0,)),
            pl.BlockSpec((O_, H_), lambda i: (0, 0)),
            pl.BlockSpec((O_,), lambda i: (0,)),
        ],
        out_specs=pl.BlockSpec((RBG, O_), lambda i: (i, 0)),
        out_shape=jax.ShapeDtypeStruct((NP, O_), jnp.float32),
    )(s2, z2s.reshape(NQ, T_, NP, HQ),
      dinv_g, b2, W_ih, W_hh, b_ih, b_hh, fcW, fcb)

    return out[:N_, 0], out[:N_, 1]


# speedup vs baseline: 9.4019x; 2.1520x over previous
import jax
import jax.numpy as jnp
from jax import lax
from jax.experimental import pallas as pl
from jax.experimental.pallas import tpu as pltpu
from jax.experimental.pallas import tpu_sc as plsc

N_, T_, F_, H_, E_, O_ = 10000, 12, 128, 128, 320000, 2
NC, NS = 2, 16
NW = NC * NS
NP = 10240
NQ = 4
HQ = H_ // NQ
K_ = 128
EPS = E_ // NS
NCH2 = -(-EPS // K_)
EPP2 = NCH2 * K_
RPS = NP // NS
RZB = 128
RB = 1024
NB = NP // RB
RBG = 512
NBG = NP // RBG
MSK = (1 << 14) - 1
NCHE = -(-NCH2 // 4) * 4


def _deg_body(pk, ewp, out, pk_v, ew_v, deg_v):
    c = lax.axis_index("c")
    s = lax.axis_index("s")
    wid = s * NC + c
    pltpu.sync_copy(pk.at[s], pk_v)
    pltpu.sync_copy(ewp.at[s], ew_v)

    def zero(i, _):
        deg_v[pl.ds(i * 16, 16)] = jnp.zeros((16,), jnp.float32)
        return 0

    lax.fori_loop(0, NP // 16, zero, 0)

    def chunk(jj, _):
        j = jj * NC + c
        for q in range(8):
            sl = pl.ds(q * 16, 16)
            d = lax.shift_right_logical(pk_v[j, sl], 14)
            plsc.addupdate_scatter(deg_v, [d], ew_v[pl.ds(j * K_ + q * 16, 16)])
        return 0

    lax.fori_loop(0, (NCH2 + 1 - c) // NC, chunk, 0)
    pltpu.sync_copy(deg_v, out.at[pl.ds(wid * NP, NP)])


def _spmm_body(table, pk, ewp, out, pk_v, dst_v, ew_v, idx_v,
               rows_r, zero_v, acc,
               gs0, gs1, gs2, gs3, ss0, ss1, ss2, ss3):
    gsem = (gs0, gs1, gs2, gs3)
    ssem = (ss0, ss1, ss2, ss3)
    c = lax.axis_index("c")
    s = lax.axis_index("s")
    pltpu.sync_copy(pk.at[s], pk_v)
    pltpu.sync_copy(ewp.at[s], ew_v)

    @plsc.parallel_loop(0, NCH2, 1, unroll=4)
    def upk(j):
        for q in range(8):
            sl = pl.ds(q * 16, 16)
            dst_v[j, sl] = lax.shift_right_logical(pk_v[j, sl], 14)

    def zb(i, _):
        for q in range(HQ // 16):
            zero_v[i, pl.ds(q * 16, 16)] = jnp.zeros((16,), jnp.float32)
        return 0

    lax.fori_loop(0, RZB, zb, 0)
    for i in range(RPS // RZB):
        pltpu.sync_copy(zero_v, acc.at[pl.ds(s * RPS + i * RZB, RZB)])

    def u_body(u, _):
        t = u // 2
        qid = c + 2 * (u % 2)
        off = qid * (T_ * NP) + t * NP

        @plsc.parallel_loop(0, NCH2, 1, unroll=4)
        def ib(j):
            for q in range(8):
                sl = pl.ds(q * 16, 16)
                idx_v[j, sl] = (pk_v[j, sl] & MSK) + off
        plsc.subcore_barrier()
        pltpu.async_copy(table.at[idx_v.at[0]], rows_r.at[0], gsem[0])
        pltpu.async_copy(table.at[idx_v.at[1]], rows_r.at[1], gsem[1])

        @pl.loop(0, NCHE, step=4)
        def grp(j0):
            for b in range(4):
                jj = j0 + b
                bp = (b + 2) % 4

                @pl.when(jj + 2 < NCH2)
                def _():
                    @pl.when(jj >= 2)
                    def _():
                        pltpu.make_async_copy(
                            table.at[pl.ds(0, K_)], rows_r.at[bp],
                            ssem[bp]).wait()
                    pltpu.async_copy(table.at[idx_v.at[jj + 2]],
                                     rows_r.at[bp], gsem[bp])

                @pl.when(jj < NCH2)
                def _():
                    pltpu.make_async_copy(
                        table.at[pl.ds(0, K_)], rows_r.at[b],
                        gsem[b]).wait()
                    rows_v = rows_r.at[b]
                    base = jj * K_

                    @plsc.parallel_loop(0, K_, 1, unroll=8)
                    def scale(e):
                        nv = plsc.load_gather(
                            ew_v, [jnp.full((16,), base + e, jnp.int32)])
                        for q in range(HQ // 16):
                            sl = pl.ds(q * 16, 16)
                            rows_v[e, sl] = rows_v[e, sl] * nv

                    pltpu.async_copy(rows_r.at[b], acc.at[dst_v.at[jj]],
                                     ssem[b], add=True)

        for b in range(4):
            pltpu.make_async_copy(table.at[pl.ds(0, K_)], rows_r.at[b],
                                  ssem[b]).wait()
        plsc.subcore_barrier()
        for i in range(RPS // RZB):
            sl = pl.ds(s * RPS + i * RZB, RZB)
            pltpu.sync_copy(acc.at[sl], out.at[qid].at[t].at[sl])
            pltpu.sync_copy(zero_v, acc.at[sl])
        return 0

    lax.fori_loop(0, T_ * 2, u_body, 0)


def _dinv_tc(parts_ref, o_ref):
    deg = jnp.sum(parts_ref[...], axis=0) + 1.0
    safe = jnp.where(deg > 0, deg, 1.0)
    o_ref[...] = jnp.where(deg > 0, lax.rsqrt(safe), 0.0)


def _mm1_tc(x_ref, w_ref, dinv_ref, o_ref):
    z = lax.dot_general(x_ref[...], w_ref[...],
                        (((1,), (1,)), ((), ())),
                        preferred_element_type=jnp.float32)
    z = z * dinv_ref[0, 0, :][:, None]
    for q in range(NQ):
        o_ref[q] = z[:, q * HQ:(q + 1) * HQ]


def _mm2_tc(s_ref, z_ref, dinv_ref, b_ref, w_ref, o_ref):
    d = dinv_ref[0, 0, :][:, None]
    pre = jnp.concatenate(
        [s_ref[q] + z_ref[q] for q in range(NQ)], axis=1)
    h = jnp.maximum(d * pre + b_ref[...][None, :], 0.0)
    z = d * lax.dot_general(h, w_ref[...], (((1,), (1,)), ((), ())),
                            preferred_element_type=jnp.float32)
    for q in range(NQ):
        o_ref[q] = z[:, q * HQ:(q + 1) * HQ]


def _gru_tc(s_ref, z_ref, dinv_ref, b2_ref,
            wih_ref, whh_ref, bih_ref, bhh_ref, fcw_ref, fcb_ref, o_ref):
    d = dinv_ref[0, 0, :][:, None]
    b2 = b2_ref[...][None, :]
    bih = bih_ref[...][None, :]
    bhh = bhh_ref[...][None, :]
    wih = wih_ref[...]
    whh = whh_ref[...]
    h = jnp.zeros((RBG, H_), jnp.float32)
    for t in range(T_):
        pre = jnp.concatenate(
            [s_ref[q, t] + z_ref[q, t] for q in range(NQ)], axis=1)
        h2 = jnp.maximum(d * pre + b2, 0.0)
        gi = lax.dot_general(h2, wih, (((1,), (1,)), ((), ())),
                             preferred_element_type=jnp.float32) + bih
        gh = lax.dot_general(h, whh, (((1,), (1,)), ((), ())),
                             preferred_element_type=jnp.float32) + bhh
        r = jax.nn.sigmoid(gi[:, :H_] + gh[:, :H_])
        zg = jax.nn.sigmoid(gi[:, H_:2 * H_] + gh[:, H_:2 * H_])
        ng = jnp.tanh(gi[:, 2 * H_:] + r * gh[:, 2 * H_:])
        h = (1.0 - zg) * ng + zg * h
    preds = lax.dot_general(h, fcw_ref[...], (((1,), (1,)), ((), ())),
                            preferred_element_type=jnp.float32)
    preds = preds + fcb_ref[...][None, :]
    mu = jax.nn.sigmoid(preds[:, 0:1])
    sg = jax.nn.softplus(preds[:, 1:2]) + 1e-6
    o_ref[...] = jnp.concatenate([mu, sg], axis=1)


def kernel(x, edge_index, edge_weight, W1, b1, W2, b2,
           W_ih, W_hh, b_ih, b_hh, fcW, fcb):
    src = edge_index[0].astype(jnp.int32)
    dst = edge_index[1].astype(jnp.int32)
    ew = edge_weight.astype(jnp.float32)
    packed = src | (dst << 14)
    spad = EPP2 - EPS
    pk2 = jnp.pad(packed.reshape(NS, EPS), ((0, 0), (0, spad))).reshape(NS, NCH2, K_)
    ewp2 = jnp.pad(ew.reshape(NS, EPS), ((0, 0), (0, spad)))
    xT = jnp.pad(jnp.transpose(x, (1, 0, 2)),
                 ((0, 0), (0, NP - N_), (0, 0))).reshape(T_ * NP, F_)

    mesh = plsc.VectorSubcoreMesh(core_axis_name="c", subcore_axis_name="s")
    deg_fn = pl.kernel(
        _deg_body,
        out_type=jax.ShapeDtypeStruct((NW * NP,), jnp.float32),
        mesh=mesh,
        compiler_params=pltpu.CompilerParams(needs_layout_passes=False,
                                             use_tc_tiling_on_sc=False),
        scratch_types=[
            pltpu.VMEM((NCH2, K_), jnp.int32),
            pltpu.VMEM((EPP2,), jnp.float32),
            pltpu.VMEM((NP,), jnp.float32),
        ],
    )
    degp = deg_fn(pk2, ewp2)

    dinv = pl.pallas_call(
        _dinv_tc,
        out_shape=jax.ShapeDtypeStruct((NP,), jnp.float32),
    )(degp.reshape(NW, NP))
    dinv3 = dinv.reshape(NB, 1, RB)

    z1s = pl.pallas_call(
        _mm1_tc,
        grid=(T_ * NP // RB,),
        in_specs=[
            pl.BlockSpec((RB, F_), lambda i: (i, 0)),
            pl.BlockSpec((H_, F_), lambda i: (0, 0)),
            pl.BlockSpec((1, 1, RB), lambda i: (i % NB, 0, 0)),
        ],
        out_specs=pl.BlockSpec((NQ, RB, HQ), lambda i: (0, i, 0)),
        out_shape=jax.ShapeDtypeStruct((NQ, T_ * NP, HQ), jnp.float32),
    )(xT, W1, dinv3)

    spmm_fn = pl.kernel(
        _spmm_body,
        out_type=jax.ShapeDtypeStruct((NQ, T_, NP, HQ), jnp.float32),
        mesh=mesh,
        compiler_params=pltpu.CompilerParams(needs_layout_passes=False,
                                             use_tc_tiling_on_sc=False),
        scratch_types=[
            pltpu.VMEM((NCH2, K_), jnp.int32),
            pltpu.VMEM((NCH2, K_), jnp.int32),
            pltpu.VMEM((EPP2,), jnp.float32),
            pltpu.VMEM((NCH2, K_), jnp.int32),
            pltpu.VMEM((4, K_, HQ), jnp.float32),
            pltpu.VMEM((RZB, HQ), jnp.float32),
            pltpu.VMEM_SHARED((NP, HQ), jnp.float32),
            pltpu.SemaphoreType.DMA, pltpu.SemaphoreType.DMA,
            pltpu.SemaphoreType.DMA, pltpu.SemaphoreType.DMA,
            pltpu.SemaphoreType.DMA, pltpu.SemaphoreType.DMA,
            pltpu.SemaphoreType.DMA, pltpu.SemaphoreType.DMA,
        ],
    )

    s1 = spmm_fn(z1s.reshape(NQ * T_ * NP, HQ), pk2, ewp2)

    z2s = pl.pallas_call(
        _mm2_tc,
        grid=(T_ * NP // RB,),
        in_specs=[
            pl.BlockSpec((NQ, RB, HQ), lambda i: (0, i, 0)),
            pl.BlockSpec((NQ, RB, HQ), lambda i: (0, i, 0)),
            pl.BlockSpec((1, 1, RB), lambda i: (i % NB, 0, 0)),
            pl.BlockSpec((H_,), lambda i: (0,)),
            pl.BlockSpec((H_, H_), lambda i: (0, 0)),
        ],
        out_specs=pl.BlockSpec((NQ, RB, HQ), lambda i: (0, i, 0)),
        out_shape=jax.ShapeDtypeStruct((NQ, T_ * NP, HQ), jnp.float32),
    )(s1.reshape(NQ, T_ * NP, HQ), z1s, dinv3, b1, W2)

    s2 = spmm_fn(z2s.reshape(NQ * T_ * NP, HQ), pk2, ewp2)

    dinv_g = dinv.reshape(NBG, 1, RBG)
    out = pl.pallas_call(
        _gru_tc,
        grid=(NBG,),
        in_specs=[
            pl.BlockSpec((NQ, T_, RBG, HQ), lambda i: (0, 0, i, 0)),
            pl.BlockSpec((NQ, T_, RBG, HQ), lambda i: (0, 0, i, 0)),
            pl.BlockSpec((1, 1, RBG), lambda i: (i, 0, 0)),
            pl.BlockSpec((H_,), lambda i: (0,)),
            pl.BlockSpec((3 * H_, H_), lambda i: (0, 0)),
            pl.BlockSpec((3 * H_, H_), lambda i: (0, 0)),
            pl.BlockSpec((3 * H_,), lambda i: (0,)),
            pl.BlockSpec((3 * H_,), lambda i: (0,)),
            pl.BlockSpec((O_, H_), lambda i: (0, 0)),
            pl.BlockSpec((O_,), lambda i: (0,)),
        ],
        out_specs=pl.BlockSpec((RBG, O_), lambda i: (i, 0)),
        out_shape=jax.ShapeDtypeStruct((NP, O_), jnp.float32),
    )(s2, z2s.reshape(NQ, T_, NP, HQ),
      dinv_g, b2, W_ih, W_hh, b_ih, b_hh, fcW, fcb)

    return out[:N_, 0], out[:N_, 1]


# speedup vs baseline: 9.5994x; 1.0210x over previous
import jax
import jax.numpy as jnp
import numpy as np
from jax import lax
from jax.experimental import pallas as pl
from jax.experimental.pallas import tpu as pltpu
from jax.experimental.pallas import tpu_sc as plsc

N_, T_, F_, H_, E_, O_ = 10000, 12, 128, 128, 320000, 2
NC, NS = 2, 16
NW = NC * NS
NP = 10240
NQ = 4
HQ = H_ // NQ
K_ = 128
EPS = E_ // NS
NCH2 = -(-EPS // K_)
EPP2 = NCH2 * K_
RPS = NP // NS
RZB = 128
RB = 1024
NB = NP // RB
RBG = 512
NBG = NP // RBG
MSK = (1 << 14) - 1
NCHE = -(-NCH2 // 4) * 4

_GDN = lax.GatherDimensionNumbers(
    offset_dims=(), collapsed_slice_dims=(0,), start_index_map=(0,))


def _deg_body(pk, ewp, out, pk_v, ew_v, deg_v):
    c = lax.axis_index("c")
    s = lax.axis_index("s")
    wid = s * NC + c
    pltpu.sync_copy(pk.at[s], pk_v)
    pltpu.sync_copy(ewp.at[s], ew_v)

    def zero(i, _):
        deg_v[pl.ds(i * 16, 16)] = jnp.zeros((16,), jnp.float32)
        return 0

    lax.fori_loop(0, NP // 16, zero, 0)

    def chunk(jj, _):
        j = jj * NC + c
        for q in range(8):
            sl = pl.ds(q * 16, 16)
            d = lax.shift_right_logical(pk_v[j, sl], 14)
            plsc.addupdate_scatter(deg_v, [d], ew_v[pl.ds(j * K_ + q * 16, 16)])
        return 0

    lax.fori_loop(0, (NCH2 + 1 - c) // NC, chunk, 0)
    pltpu.sync_copy(deg_v, out.at[pl.ds(wid * NP, NP)])


def _spmm_body(table, pk, ewp, out, pk_v, dst_v, ew_v, idx_v,
               rows_r, zero_v, acc,
               gs0, gs1, gs2, gs3, ss0, ss1, ss2, ss3):
    gsem = (gs0, gs1, gs2, gs3)
    ssem = (ss0, ss1, ss2, ss3)
    c = lax.axis_index("c")
    s = lax.axis_index("s")
    pltpu.sync_copy(pk.at[s], pk_v)
    pltpu.sync_copy(ewp.at[s], ew_v)

    @plsc.parallel_loop(0, NCH2, 1, unroll=4)
    def upk(j):
        for q in range(8):
            sl = pl.ds(q * 16, 16)
            dst_v[j, sl] = lax.shift_right_logical(pk_v[j, sl], 14)

    def zb(i, _):
        for q in range(HQ // 16):
            zero_v[i, pl.ds(q * 16, 16)] = jnp.zeros((16,), jnp.float32)
        return 0

    lax.fori_loop(0, RZB, zb, 0)
    for i in range(RPS // RZB):
        pltpu.sync_copy(zero_v, acc.at[pl.ds(s * RPS + i * RZB, RZB)])

    def u_body(u, _):
        t = u // 2
        qid = c + 2 * (u % 2)
        off = qid * (T_ * NP) + t * NP

        @plsc.parallel_loop(0, NCH2, 1, unroll=4)
        def ib(j):
            for q in range(8):
                sl = pl.ds(q * 16, 16)
                idx_v[j, sl] = (pk_v[j, sl] & MSK) + off
        plsc.subcore_barrier()
        pltpu.async_copy(table.at[idx_v.at[0]], rows_r.at[0], gsem[0])
        pltpu.async_copy(table.at[idx_v.at[1]], rows_r.at[1], gsem[1])

        @pl.loop(0, NCHE, step=4)
        def grp(j0):
            for b in range(4):
                jj = j0 + b
                bp = (b + 2) % 4

                @pl.when(jj + 2 < NCH2)
                def _():
                    @pl.when(jj >= 2)
                    def _():
                        pltpu.make_async_copy(
                            table.at[pl.ds(0, K_)], rows_r.at[bp],
                            ssem[bp]).wait()
                    pltpu.async_copy(table.at[idx_v.at[jj + 2]],
                                     rows_r.at[bp], gsem[bp])

                @pl.when(jj < NCH2)
                def _():
                    pltpu.make_async_copy(
                        table.at[pl.ds(0, K_)], rows_r.at[b],
                        gsem[b]).wait()
                    rows_v = rows_r.at[b]
                    base = jj * K_

                    @plsc.parallel_loop(0, K_ // 16, 1, unroll=2)
                    def scale(g):
                        ew16 = ew_v[pl.ds(base + g * 16, 16)]
                        for lane in range(16):
                            e = g * 16 + lane
                            nv = lax.gather(
                                ew16, jnp.full((16, 1), lane, jnp.int32),
                                dimension_numbers=_GDN, slice_sizes=(1,),
                                mode=lax.GatherScatterMode.PROMISE_IN_BOUNDS)
                            for q in range(HQ // 16):
                                sl = pl.ds(q * 16, 16)
                                rows_v[e, sl] = rows_v[e, sl] * nv

                    pltpu.async_copy(rows_r.at[b], acc.at[dst_v.at[jj]],
                                     ssem[b], add=True)

        for b in range(4):
            pltpu.make_async_copy(table.at[pl.ds(0, K_)], rows_r.at[b],
                                  ssem[b]).wait()
        plsc.subcore_barrier()
        for i in range(RPS // RZB):
            sl = pl.ds(s * RPS + i * RZB, RZB)
            pltpu.sync_copy(acc.at[sl], out.at[qid].at[t].at[sl])
            pltpu.sync_copy(zero_v, acc.at[sl])
        return 0

    lax.fori_loop(0, T_ * 2, u_body, 0)


def _dinv_tc(parts_ref, o_ref):
    deg = jnp.sum(parts_ref[...], axis=0) + 1.0
    safe = jnp.where(deg > 0, deg, 1.0)
    o_ref[...] = jnp.where(deg > 0, lax.rsqrt(safe), 0.0)


def _mm1_tc(x_ref, w_ref, dinv_ref, o_ref):
    z = lax.dot_general(x_ref[...], w_ref[...],
                        (((1,), (1,)), ((), ())),
                        preferred_element_type=jnp.float32)
    z = z * dinv_ref[0, 0, :][:, None]
    for q in range(NQ):
        o_ref[q] = z[:, q * HQ:(q + 1) * HQ]


def _mm2_tc(s_ref, z_ref, dinv_ref, b_ref, w_ref, o_ref):
    d = dinv_ref[0, 0, :][:, None]
    pre = jnp.concatenate(
        [s_ref[q] + z_ref[q] for q in range(NQ)], axis=1)
    h = jnp.maximum(d * pre + b_ref[...][None, :], 0.0)
    z = d * lax.dot_general(h, w_ref[...], (((1,), (1,)), ((), ())),
                            preferred_element_type=jnp.float32)
    for q in range(NQ):
        o_ref[q] = z[:, q * HQ:(q + 1) * HQ]


def _gru_tc(s_ref, z_ref, dinv_ref, b2_ref,
            wih_ref, whh_ref, bih_ref, bhh_ref, fcw_ref, fcb_ref, o_ref):
    d = dinv_ref[0, 0, :][:, None]
    b2 = b2_ref[...][None, :]
    bih = bih_ref[...][None, :]
    bhh = bhh_ref[...][None, :]
    wih = wih_ref[...]
    whh = whh_ref[...]
    h = jnp.zeros((RBG, H_), jnp.float32)
    for t in range(T_):
        pre = jnp.concatenate(
            [s_ref[q, t] + z_ref[q, t] for q in range(NQ)], axis=1)
        h2 = jnp.maximum(d * pre + b2, 0.0)
        gi = lax.dot_general(h2, wih, (((1,), (1,)), ((), ())),
                             preferred_element_type=jnp.float32) + bih
        gh = lax.dot_general(h, whh, (((1,), (1,)), ((), ())),
                             preferred_element_type=jnp.float32) + bhh
        r = jax.nn.sigmoid(gi[:, :H_] + gh[:, :H_])
        zg = jax.nn.sigmoid(gi[:, H_:2 * H_] + gh[:, H_:2 * H_])
        ng = jnp.tanh(gi[:, 2 * H_:] + r * gh[:, 2 * H_:])
        h = (1.0 - zg) * ng + zg * h
    preds = lax.dot_general(h, fcw_ref[...], (((1,), (1,)), ((), ())),
                            preferred_element_type=jnp.float32)
    preds = preds + fcb_ref[...][None, :]
    mu = jax.nn.sigmoid(preds[:, 0:1])
    sg = jax.nn.softplus(preds[:, 1:2]) + 1e-6
    o_ref[...] = jnp.concatenate([mu, sg], axis=1)


def kernel(x, edge_index, edge_weight, W1, b1, W2, b2,
           W_ih, W_hh, b_ih, b_hh, fcW, fcb):
    src = edge_index[0].astype(jnp.int32)
    dst = edge_index[1].astype(jnp.int32)
    ew = edge_weight.astype(jnp.float32)
    packed = src | (dst << 14)
    spad = EPP2 - EPS
    pk2 = jnp.pad(packed.reshape(NS, EPS), ((0, 0), (0, spad))).reshape(NS, NCH2, K_)
    ewp2 = jnp.pad(ew.reshape(NS, EPS), ((0, 0), (0, spad)))
    xT = jnp.pad(jnp.transpose(x, (1, 0, 2)),
                 ((0, 0), (0, NP - N_), (0, 0))).reshape(T_ * NP, F_)

    mesh = plsc.VectorSubcoreMesh(core_axis_name="c", subcore_axis_name="s")
    deg_fn = pl.kernel(
        _deg_body,
        out_type=jax.ShapeDtypeStruct((NW * NP,), jnp.float32),
        mesh=mesh,
        compiler_params=pltpu.CompilerParams(needs_layout_passes=False,
                                             use_tc_tiling_on_sc=False),
        scratch_types=[
            pltpu.VMEM((NCH2, K_), jnp.int32),
            pltpu.VMEM((EPP2,), jnp.float32),
            pltpu.VMEM((NP,), jnp.float32),
        ],
    )
    degp = deg_fn(pk2, ewp2)

    dinv = pl.pallas_call(
        _dinv_tc,
        out_shape=jax.ShapeDtypeStruct((NP,), jnp.float32),
    )(degp.reshape(NW, NP))
    dinv3 = dinv.reshape(NB, 1, RB)

    z1s = pl.pallas_call(
        _mm1_tc,
        grid=(T_ * NP // RB,),
        in_specs=[
            pl.BlockSpec((RB, F_), lambda i: (i, 0)),
            pl.BlockSpec((H_, F_), lambda i: (0, 0)),
            pl.BlockSpec((1, 1, RB), lambda i: (i % NB, 0, 0)),
        ],
        out_specs=pl.BlockSpec((NQ, RB, HQ), lambda i: (0, i, 0)),
        out_shape=jax.ShapeDtypeStruct((NQ, T_ * NP, HQ), jnp.float32),
    )(xT, W1, dinv3)

    spmm_fn = pl.kernel(
        _spmm_body,
        out_type=jax.ShapeDtypeStruct((NQ, T_, NP, HQ), jnp.float32),
        mesh=mesh,
        compiler_params=pltpu.CompilerParams(needs_layout_passes=False,
                                             use_tc_tiling_on_sc=False),
        scratch_types=[
            pltpu.VMEM((NCH2, K_), jnp.int32),
            pltpu.VMEM((NCH2, K_), jnp.int32),
            pltpu.VMEM((EPP2,), jnp.float32),
            pltpu.VMEM((NCH2, K_), jnp.int32),
            pltpu.VMEM((4, K_, HQ), jnp.float32),
            pltpu.VMEM((RZB, HQ), jnp.float32),
            pltpu.VMEM_SHARED((NP, HQ), jnp.float32),
            pltpu.SemaphoreType.DMA, pltpu.SemaphoreType.DMA,
            pltpu.SemaphoreType.DMA, pltpu.SemaphoreType.DMA,
            pltpu.SemaphoreType.DMA, pltpu.SemaphoreType.DMA,
            pltpu.SemaphoreType.DMA, pltpu.SemaphoreType.DMA,
        ],
    )

    s1 = spmm_fn(z1s.reshape(NQ * T_ * NP, HQ), pk2, ewp2)

    z2s = pl.pallas_call(
        _mm2_tc,
        grid=(T_ * NP // RB,),
        in_specs=[
            pl.BlockSpec((NQ, RB, HQ), lambda i: (0, i, 0)),
            pl.BlockSpec((NQ, RB, HQ), lambda i: (0, i, 0)),
            pl.BlockSpec((1, 1, RB), lambda i: (i % NB, 0, 0)),
            pl.BlockSpec((H_,), lambda i: (0,)),
            pl.BlockSpec((H_, H_), lambda i: (0, 0)),
        ],
        out_specs=pl.BlockSpec((NQ, RB, HQ), lambda i: (0, i, 0)),
        out_shape=jax.ShapeDtypeStruct((NQ, T_ * NP, HQ), jnp.float32),
    )(s1.reshape(NQ, T_ * NP, HQ), z1s, dinv3, b1, W2)

    s2 = spmm_fn(z2s.reshape(NQ * T_ * NP, HQ), pk2, ewp2)

    dinv_g = dinv.reshape(NBG, 1, RBG)
    out = pl.pallas_call(
        _gru_tc,
        grid=(NBG,),
        in_specs=[
            pl.BlockSpec((NQ, T_, RBG, HQ), lambda i: (0, 0, i, 0)),
            pl.BlockSpec((NQ, T_, RBG, HQ), lambda i: (0, 0, i, 0)),
            pl.BlockSpec((1, 1, RBG), lambda i: (i, 0, 0)),
            pl.BlockSpec((H_,), lambda i: (0,)),
            pl.BlockSpec((3 * H_, H_), lambda i: (0, 0)),
            pl.BlockSpec((3 * H_, H_), lambda i: (0, 0)),
            pl.BlockSpec((3 * H_,), lambda i: (0,)),
            pl.BlockSpec((3 * H_,), lambda i: (0,)),
            pl.BlockSpec((O_, H_), lambda i: (0, 0)),
            pl.BlockSpec((O_,), lambda i: (0,)),
        ],
        out_specs=pl.BlockSpec((RBG, O_), lambda i: (i, 0)),
        out_shape=jax.ShapeDtypeStruct((NP, O_), jnp.float32),
    )(s2, z2s.reshape(NQ, T_, NP, HQ),
      dinv_g, b2, W_ih, W_hh, b_ih, b_hh, fcW, fcb)

    return out[:N_, 0], out[:N_, 1]


# speedup vs baseline: 9.7160x; 1.0121x over previous
import jax
import jax.numpy as jnp
import numpy as np
from jax import lax
from jax.experimental import pallas as pl
from jax.experimental.pallas import tpu as pltpu
from jax.experimental.pallas import tpu_sc as plsc

N_, T_, F_, H_, E_, O_ = 10000, 12, 128, 128, 320000, 2
NC, NS = 2, 16
NW = NC * NS
NP = 10240
NQ = 4
HQ = H_ // NQ
K_ = 128
EPS = E_ // NS
NCH2 = -(-EPS // K_)
EPP2 = NCH2 * K_
RPS = NP // NS
RZB = 128
RB = 1024
NB = NP // RB
RB1 = 1000
NB1 = N_ // RB1
RBG = 400
NBG = N_ // RBG
MSK = (1 << 14) - 1
NCHE = -(-NCH2 // 4) * 4

_GDN = lax.GatherDimensionNumbers(
    offset_dims=(), collapsed_slice_dims=(0,), start_index_map=(0,))


def _deg_body(pk, ewp, out, pk_v, ew_v, deg_v):
    c = lax.axis_index("c")
    s = lax.axis_index("s")
    wid = s * NC + c
    pltpu.sync_copy(pk.at[s], pk_v)
    pltpu.sync_copy(ewp.at[s], ew_v)

    def zero(i, _):
        deg_v[pl.ds(i * 16, 16)] = jnp.zeros((16,), jnp.float32)
        return 0

    lax.fori_loop(0, NP // 16, zero, 0)

    def chunk(jj, _):
        j = jj * NC + c
        for q in range(8):
            sl = pl.ds(q * 16, 16)
            d = lax.shift_right_logical(pk_v[j, sl], 14)
            plsc.addupdate_scatter(deg_v, [d], ew_v[pl.ds(j * K_ + q * 16, 16)])
        return 0

    lax.fori_loop(0, (NCH2 + 1 - c) // NC, chunk, 0)
    pltpu.sync_copy(deg_v, out.at[pl.ds(wid * NP, NP)])


def _spmm_body(table, pk, ewp, out, pk_v, dst_v, ew_v, idx_v,
               rows_r, zero_v, acc,
               gs0, gs1, gs2, gs3, ss0, ss1, ss2, ss3):
    gsem = (gs0, gs1, gs2, gs3)
    ssem = (ss0, ss1, ss2, ss3)
    c = lax.axis_index("c")
    s = lax.axis_index("s")
    pltpu.sync_copy(pk.at[s], pk_v)
    pltpu.sync_copy(ewp.at[s], ew_v)

    @plsc.parallel_loop(0, NCH2, 1, unroll=4)
    def upk(j):
        for q in range(8):
            sl = pl.ds(q * 16, 16)
            dst_v[j, sl] = lax.shift_right_logical(pk_v[j, sl], 14)

    def zb(i, _):
        for q in range(HQ // 16):
            zero_v[i, pl.ds(q * 16, 16)] = jnp.zeros((16,), jnp.float32)
        return 0

    lax.fori_loop(0, RZB, zb, 0)
    for i in range(RPS // RZB):
        pltpu.sync_copy(zero_v, acc.at[pl.ds(s * RPS + i * RZB, RZB)])

    def u_body(u, _):
        t = u // 2
        qid = c + 2 * (u % 2)
        off = qid * (T_ * NP) + t * NP

        @plsc.parallel_loop(0, NCH2, 1, unroll=4)
        def ib(j):
            for q in range(8):
                sl = pl.ds(q * 16, 16)
                idx_v[j, sl] = (pk_v[j, sl] & MSK) + off
        plsc.subcore_barrier()
        pltpu.async_copy(table.at[idx_v.at[0]], rows_r.at[0], gsem[0])
        pltpu.async_copy(table.at[idx_v.at[1]], rows_r.at[1], gsem[1])

        @pl.loop(0, NCHE, step=4)
        def grp(j0):
            for b in range(4):
                jj = j0 + b
                bp = (b + 2) % 4

                @pl.when(jj + 2 < NCH2)
                def _():
                    @pl.when(jj >= 2)
                    def _():
                        pltpu.make_async_copy(
                            table.at[pl.ds(0, K_)], rows_r.at[bp],
                            ssem[bp]).wait()
                    pltpu.async_copy(table.at[idx_v.at[jj + 2]],
                                     rows_r.at[bp], gsem[bp])

                @pl.when(jj < NCH2)
                def _():
                    pltpu.make_async_copy(
                        table.at[pl.ds(0, K_)], rows_r.at[b],
                        gsem[b]).wait()
                    rows_v = rows_r.at[b]
                    base = jj * K_

                    @plsc.parallel_loop(0, K_ // 16, 1, unroll=2)
                    def scale(g):
                        ew16 = ew_v[pl.ds(base + g * 16, 16)]
                        for lane in range(16):
                            e = g * 16 + lane
                            nv = lax.gather(
                                ew16, jnp.full((16, 1), lane, jnp.int32),
                                dimension_numbers=_GDN, slice_sizes=(1,),
                                mode=lax.GatherScatterMode.PROMISE_IN_BOUNDS)
                            for q in range(HQ // 16):
                                sl = pl.ds(q * 16, 16)
                                rows_v[e, sl] = rows_v[e, sl] * nv

                    pltpu.async_copy(rows_r.at[b], acc.at[dst_v.at[jj]],
                                     ssem[b], add=True)

        for b in range(4):
            pltpu.make_async_copy(table.at[pl.ds(0, K_)], rows_r.at[b],
                                  ssem[b]).wait()
        plsc.subcore_barrier()
        for i in range(RPS // RZB):
            sl = pl.ds(s * RPS + i * RZB, RZB)
            pltpu.sync_copy(acc.at[sl], out.at[qid].at[t].at[sl])
            pltpu.sync_copy(zero_v, acc.at[sl])
        return 0

    lax.fori_loop(0, T_ * 2, u_body, 0)


def _dinv_tc(parts_ref, o_ref):
    deg = jnp.sum(parts_ref[...], axis=0) + 1.0
    safe = jnp.where(deg > 0, deg, 1.0)
    o_ref[...] = jnp.where(deg > 0, lax.rsqrt(safe), 0.0)


def _mm1_tc(x_ref, w_ref, dinv_ref, o_ref):
    z = lax.dot_general(x_ref[...], w_ref[...],
                        (((1,), (1,)), ((), ())),
                        preferred_element_type=jnp.float32)
    z = z * dinv_ref[0, 0, :][:, None]
    for q in range(NQ):
        o_ref[q, 0] = z[:, q * HQ:(q + 1) * HQ]


def _mm2_tc(s_ref, z_ref, dinv_ref, b_ref, w_ref, o_ref):
    d = dinv_ref[0, 0, :][:, None]
    pre = jnp.concatenate(
        [s_ref[q, 0] + z_ref[q, 0] for q in range(NQ)], axis=1)
    h = jnp.maximum(d * pre + b_ref[...][None, :], 0.0)
    z = d * lax.dot_general(h, w_ref[...], (((1,), (1,)), ((), ())),
                            preferred_element_type=jnp.float32)
    for q in range(NQ):
        o_ref[q, 0] = z[:, q * HQ:(q + 1) * HQ]


def _gru_tc(s_ref, z_ref, dinv_ref, b2_ref,
            wih_ref, whh_ref, bih_ref, bhh_ref, fcw_ref, fcb_ref, o_ref):
    d = dinv_ref[0, 0, :][:, None]
    b2 = b2_ref[...][None, :]
    bih = bih_ref[...][None, :]
    bhh = bhh_ref[...][None, :]
    wih = wih_ref[...]
    whh = whh_ref[...]
    h = jnp.zeros((RBG, H_), jnp.float32)
    for t in range(T_):
        pre = jnp.concatenate(
            [s_ref[q, t] + z_ref[q, t] for q in range(NQ)], axis=1)
        h2 = jnp.maximum(d * pre + b2, 0.0)
        gi = lax.dot_general(h2, wih, (((1,), (1,)), ((), ())),
                             preferred_element_type=jnp.float32) + bih
        gh = lax.dot_general(h, whh, (((1,), (1,)), ((), ())),
                             preferred_element_type=jnp.float32) + bhh
        r = jax.nn.sigmoid(gi[:, :H_] + gh[:, :H_])
        zg = jax.nn.sigmoid(gi[:, H_:2 * H_] + gh[:, H_:2 * H_])
        ng = jnp.tanh(gi[:, 2 * H_:] + r * gh[:, 2 * H_:])
        h = (1.0 - zg) * ng + zg * h
    preds = lax.dot_general(h, fcw_ref[...], (((1,), (1,)), ((), ())),
                            preferred_element_type=jnp.float32)
    preds = preds + fcb_ref[...][None, :]
    mu = jax.nn.sigmoid(preds[:, 0:1])
    sg = jax.nn.softplus(preds[:, 1:2]) + 1e-6
    o_ref[...] = jnp.concatenate([mu, sg], axis=1)


def kernel(x, edge_index, edge_weight, W1, b1, W2, b2,
           W_ih, W_hh, b_ih, b_hh, fcW, fcb):
    src = edge_index[0].astype(jnp.int32)
    dst = edge_index[1].astype(jnp.int32)
    ew = edge_weight.astype(jnp.float32)
    packed = src | (dst << 14)
    spad = EPP2 - EPS
    pk2 = jnp.pad(packed.reshape(NS, EPS), ((0, 0), (0, spad))).reshape(NS, NCH2, K_)
    ewp2 = jnp.pad(ew.reshape(NS, EPS), ((0, 0), (0, spad)))
    x2 = x.reshape(N_, T_ * F_)

    mesh = plsc.VectorSubcoreMesh(core_axis_name="c", subcore_axis_name="s")
    deg_fn = pl.kernel(
        _deg_body,
        out_type=jax.ShapeDtypeStruct((NW * NP,), jnp.float32),
        mesh=mesh,
        compiler_params=pltpu.CompilerParams(needs_layout_passes=False,
                                             use_tc_tiling_on_sc=False),
        scratch_types=[
            pltpu.VMEM((NCH2, K_), jnp.int32),
            pltpu.VMEM((EPP2,), jnp.float32),
            pltpu.VMEM((NP,), jnp.float32),
        ],
    )
    degp = deg_fn(pk2, ewp2)

    dinv = pl.pallas_call(
        _dinv_tc,
        out_shape=jax.ShapeDtypeStruct((NP,), jnp.float32),
    )(degp.reshape(NW, NP))
    dinv3 = dinv[:N_].reshape(NB1, 1, RB1)

    z1s = pl.pallas_call(
        _mm1_tc,
        grid=(T_ * NB1,),
        in_specs=[
            pl.BlockSpec((RB1, F_), lambda i: (i % NB1, i // NB1)),
            pl.BlockSpec((H_, F_), lambda i: (0, 0)),
            pl.BlockSpec((1, 1, RB1), lambda i: (i % NB1, 0, 0)),
        ],
        out_specs=pl.BlockSpec((NQ, 1, RB1, HQ),
                               lambda i: (0, i // NB1, i % NB1, 0)),
        out_shape=jax.ShapeDtypeStruct((NQ, T_, NP, HQ), jnp.float32),
    )(x2, W1, dinv3)

    spmm_fn = pl.kernel(
        _spmm_body,
        out_type=jax.ShapeDtypeStruct((NQ, T_, NP, HQ), jnp.float32),
        mesh=mesh,
        compiler_params=pltpu.CompilerParams(needs_layout_passes=False,
                                             use_tc_tiling_on_sc=False),
        scratch_types=[
            pltpu.VMEM((NCH2, K_), jnp.int32),
            pltpu.VMEM((NCH2, K_), jnp.int32),
            pltpu.VMEM((EPP2,), jnp.float32),
            pltpu.VMEM((NCH2, K_), jnp.int32),
            pltpu.VMEM((4, K_, HQ), jnp.float32),
            pltpu.VMEM((RZB, HQ), jnp.float32),
            pltpu.VMEM_SHARED((NP, HQ), jnp.float32),
            pltpu.SemaphoreType.DMA, pltpu.SemaphoreType.DMA,
            pltpu.SemaphoreType.DMA, pltpu.SemaphoreType.DMA,
            pltpu.SemaphoreType.DMA, pltpu.SemaphoreType.DMA,
            pltpu.SemaphoreType.DMA, pltpu.SemaphoreType.DMA,
        ],
    )

    s1 = spmm_fn(z1s.reshape(NQ * T_ * NP, HQ), pk2, ewp2)

    z2s = pl.pallas_call(
        _mm2_tc,
        grid=(T_ * NB1,),
        in_specs=[
            pl.BlockSpec((NQ, 1, RB1, HQ),
                         lambda i: (0, i // NB1, i % NB1, 0)),
            pl.BlockSpec((NQ, 1, RB1, HQ),
                         lambda i: (0, i // NB1, i % NB1, 0)),
            pl.BlockSpec((1, 1, RB1), lambda i: (i % NB1, 0, 0)),
            pl.BlockSpec((H_,), lambda i: (0,)),
            pl.BlockSpec((H_, H_), lambda i: (0, 0)),
        ],
        out_specs=pl.BlockSpec((NQ, 1, RB1, HQ),
                               lambda i: (0, i // NB1, i % NB1, 0)),
        out_shape=jax.ShapeDtypeStruct((NQ, T_, NP, HQ), jnp.float32),
    )(s1, z1s, dinv3, b1, W2)

    s2 = spmm_fn(z2s.reshape(NQ * T_ * NP, HQ), pk2, ewp2)

    dinv_g = dinv[:N_].reshape(NBG, 1, RBG)
    out = pl.pallas_call(
        _gru_tc,
        grid=(NBG,),
        in_specs=[
            pl.BlockSpec((NQ, T_, RBG, HQ), lambda i: (0, 0, i, 0)),
            pl.BlockSpec((NQ, T_, RBG, HQ), lambda i: (0, 0, i, 0)),
            pl.BlockSpec((1, 1, RBG), lambda i: (i, 0, 0)),
            pl.BlockSpec((H_,), lambda i: (0,)),
            pl.BlockSpec((3 * H_, H_), lambda i: (0, 0)),
            pl.BlockSpec((3 * H_, H_), lambda i: (0, 0)),
            pl.BlockSpec((3 * H_,), lambda i: (0,)),
            pl.BlockSpec((3 * H_,), lambda i: (0,)),
            pl.BlockSpec((O_, H_), lambda i: (0, 0)),
            pl.BlockSpec((O_,), lambda i: (0,)),
        ],
        out_specs=pl.BlockSpec((RBG, O_), lambda i: (i, 0)),
        out_shape=jax.ShapeDtypeStruct((N_, O_), jnp.float32),
    )(s2, z2s, dinv_g, b2, W_ih, W_hh, b_ih, b_hh, fcW, fcb)

    return out[:, 0], out[:, 1]
